# baseline (speedup 1.0000x reference)
import functools

import jax
import jax.numpy as jnp
from jax import lax
from jax.experimental import pallas as pl
from jax.experimental.pallas import tpu as pltpu
from jax.experimental.pallas import tpu_sc as plsc

_MARGIN = 3.0
_DIM = 64
_L = 16
_CHUNK = 128
_SPAN = 32768
_SUB = 512
_CAP = 4096
_WCAP = 1024


def _make_phase1(batch, nent):
    info = plsc.get_sparse_core_info()
    nw = info.num_cores * info.num_subcores
    ntot = 4 * batch
    tail_w = nent % _SUB
    mesh = plsc.VectorSubcoreMesh(core_axis_name="c", subcore_axis_name="s")

    @functools.partial(
        pl.kernel,
        mesh=mesh,
        out_type=jax.ShapeDtypeStruct(((ntot + 1) * _DIM,), jnp.float32),
        compiler_params=pltpu.CompilerParams(needs_layout_passes=False),
        scratch_types=[
            pltpu.VMEM((1024,), jnp.int32),
            pltpu.VMEM((_CAP,), jnp.int32),
            pltpu.VMEM((_CAP,), jnp.int32),
            pltpu.VMEM((_WCAP,), jnp.int32),
            pltpu.VMEM((_WCAP,), jnp.int32),
            pltpu.VMEM((_DIM, _SUB), jnp.float32),
            pltpu.VMEM((_L * _DIM,), jnp.float32),
            pltpu.VMEM((max(tail_w, 1), _DIM), jnp.float32),
            pltpu.SemaphoreType.DMA,
        ],
    )
    def sweep_gather(ph, pt, nh, nt, ent_t, tail_hbm, out,
                     blk, sidx, spos, widx, wpos, sw, rowstage, tailbuf,
                     sem):
        wid = lax.axis_index("s") * info.num_cores + lax.axis_index("c")
        iota = lax.iota(jnp.int32, _L)
        zero_i = jnp.zeros((_L,), jnp.int32)
        lo = wid * _SPAN
        dump = ntot

        def scan_array(ref, ai, off0):
            def blk_body(b, off):
                pltpu.sync_copy(ref.at[pl.ds(b * 1024, 1024)], blk)

                def g_body(g, off):
                    v = blk[pl.ds(g * _L, _L)]
                    pos = iota + (ai * batch + b * 1024 + g * _L)
                    m = (v >= lo) & (v < lo + _SPAN)
                    r = plsc.cumsum(m.astype(jnp.int32))
                    dst = off + r - 1
                    plsc.store_scatter(sidx, [dst], v, mask=m)
                    plsc.store_scatter(spos, [dst], pos, mask=m)
                    return off + r[15]

                return lax.fori_loop(0, 1024 // _L, g_body, off)

            return lax.fori_loop(0, batch // 1024, blk_body, off0)

        n_span = scan_array(ph, 0, jnp.int32(0))
        n_span = scan_array(pt, 1, n_span)
        n_span = scan_array(nh, 2, n_span)
        n_span = scan_array(nt, 3, n_span)
        nv_span = (n_span + _L - 1) // _L

        def extract(base, width, carry):
            def rs_body(m, nwin):
                guard = iota < (n_span - m * _L)
                v = sidx[pl.ds(m * _L, _L)]
                p = spos[pl.ds(m * _L, _L)]
                msk = (v >= base) & (v < base + width) & guard
                r = plsc.cumsum(msk.astype(jnp.int32))
                dst = nwin + r - 1
                plsc.store_scatter(widx, [dst], v - base, mask=msk)
                plsc.store_scatter(wpos, [dst], p, mask=msk)
                return nwin + r[15]

            nwin = lax.fori_loop(0, nv_span, rs_body, jnp.int32(0))

            def ex_body(m, c):
                guard = iota < (nwin - m * _L)
                lv = jnp.clip(widx[pl.ds(m * _L, _L)], 0, _SUB - 1)
                pv = jnp.where(guard, wpos[pl.ds(m * _L, _L)], dump)
                for u in range(_L):
                    col = zero_i + lv[u]
                    for t in range(_DIM // _L):
                        rowstage[pl.ds(u * _DIM + t * _L, _L)] = (
                            plsc.load_gather(sw, [iota + t * _L, col]))
                    pltpu.async_copy(
                        rowstage.at[pl.ds(u * _DIM, _DIM)],
                        out.at[pl.ds(pl.multiple_of(pv[u] * _DIM, _DIM),
                                     _DIM)],
                        sem)
                pltpu.make_async_copy(
                    out.at[pl.ds(0, _L * _DIM)], rowstage, sem).wait()
                return c

            return lax.fori_loop(0, (nwin + _L - 1) // _L, ex_body, carry)

        n_sub = jnp.clip(nent - lo, 0, _SPAN) // _SUB

        def sub_body(s, carry):
            base = pl.multiple_of(lo + s * _SUB, _SUB)
            pltpu.sync_copy(ent_t.at[:, pl.ds(base, _SUB)], sw)
            return extract(base, _SUB, carry)

        lax.fori_loop(0, n_sub, sub_body, 0)

        tail_base = (nent // _SUB) * _SUB

        @pl.when(wid == tail_base // _SPAN)
        def _():
            pltpu.sync_copy(tail_hbm, tailbuf)

            def rs_body(m, nwin):
                guard = iota < (n_span - m * _L)
                v = sidx[pl.ds(m * _L, _L)]
                p = spos[pl.ds(m * _L, _L)]
                msk = (v >= tail_base) & guard
                r = plsc.cumsum(msk.astype(jnp.int32))
                dst = nwin + r - 1
                plsc.store_scatter(widx, [dst], v - tail_base, mask=msk)
                plsc.store_scatter(wpos, [dst], p, mask=msk)
                return nwin + r[15]

            nwin = lax.fori_loop(0, nv_span, rs_body, jnp.int32(0))

            def ex_body(m, c):
                guard = iota < (nwin - m * _L)
                lv = jnp.clip(widx[pl.ds(m * _L, _L)], 0, tail_w - 1)
                pv = jnp.where(guard, wpos[pl.ds(m * _L, _L)], dump)
                for u in range(_L):
                    for t in range(_DIM // _L):
                        rowstage[pl.ds(u * _DIM + t * _L, _L)] = (
                            tailbuf[lv[u], pl.ds(t * _L, _L)])
                    pltpu.async_copy(
                        rowstage.at[pl.ds(u * _DIM, _DIM)],
                        out.at[pl.ds(pl.multiple_of(pv[u] * _DIM, _DIM),
                                     _DIM)],
                        sem)
                pltpu.make_async_copy(
                    out.at[pl.ds(0, _L * _DIM)], rowstage, sem).wait()
                return c

            lax.fori_loop(0, (nwin + _L - 1) // _L, ex_body, 0)

    return sweep_gather


def _make_phase2(batch, nrel):
    info = plsc.get_sparse_core_info()
    nw = info.num_cores * info.num_subcores
    per_w = batch // nw
    n_chunks = per_w // _CHUNK
    ntot = 4 * batch
    mesh = plsc.VectorSubcoreMesh(core_axis_name="c", subcore_axis_name="s")

    @functools.partial(
        pl.kernel,
        mesh=mesh,
        out_type=jax.ShapeDtypeStruct((nw, _L), jnp.float32),
        compiler_params=pltpu.CompilerParams(
            use_tc_tiling_on_sc=False, needs_layout_passes=False),
        scratch_types=[
            pltpu.VMEM((_CHUNK,), jnp.int32),
            pltpu.VMEM((_CHUNK * _DIM,), jnp.float32),
            pltpu.VMEM((_CHUNK * _DIM,), jnp.float32),
            pltpu.VMEM((_CHUNK * _DIM,), jnp.float32),
            pltpu.VMEM((_CHUNK * _DIM,), jnp.float32),
            pltpu.VMEM((nrel * _DIM,), jnp.float32),
            pltpu.VMEM((_L,), jnp.float32),
            pltpu.SemaphoreType.DMA,
        ],
    )
    def score(gath, pr_hbm, rel_lin, out_hbm,
              pr_i, h_rows, t_rows, nh_rows, nt_rows, rel_l, part_v, sem):
        wid = lax.axis_index("s") * info.num_cores + lax.axis_index("c")
        lane = lax.iota(jnp.int32, _L)
        zero16 = jnp.zeros((_L,), jnp.float32)

        pltpu.sync_copy(rel_lin, rel_l)

        def chunk_body(c, part):
            base = wid * per_w + c * _CHUNK
            pltpu.sync_copy(pr_hbm.at[pl.ds(base, _CHUNK)], pr_i)
            bufs = (h_rows, t_rows, nh_rows, nt_rows)
            for ai, buf in enumerate(bufs):
                src = pl.multiple_of((ai * batch + base) * _DIM, _DIM)
                pltpu.async_copy(gath.at[pl.ds(src, _CHUNK * _DIM)], buf,
                                 sem)
            for buf in bufs:
                pltpu.make_async_copy(
                    gath.at[pl.ds(0, _CHUNK * _DIM)], buf, sem).wait()

            def g_body(g, part):
                row_off = (lane + g * _L) * _DIM
                prv = pr_i[pl.ds(g * _L, _L)] * _DIM

                def d_body(d, acc):
                    for du in range(4):
                        dd = d * 4 + du
                        idx = row_off + dd
                        rv = plsc.load_gather(rel_l, [prv + dd])
                        hv = plsc.load_gather(h_rows, [idx])
                        tv = plsc.load_gather(t_rows, [idx])
                        nhv = plsc.load_gather(nh_rows, [idx])
                        ntv = plsc.load_gather(nt_rows, [idx])
                        acc = acc + (jnp.abs(hv + rv - tv)
                                     - jnp.abs(nhv + rv - ntv))
                    return acc

                diff = lax.fori_loop(0, _DIM // 4, d_body, zero16)
                return part + jnp.maximum(diff + _MARGIN, 0.0)

            return lax.fori_loop(0, _CHUNK // _L, g_body, part)

        part = lax.fori_loop(0, n_chunks, chunk_body, zero16)
        part_v[...] = part
        pltpu.sync_copy(part_v, out_hbm.at[wid])

    return score


@jax.jit
def kernel(pos_h, pos_r, pos_t, neg_h, neg_t, ent_emb, rel_emb):
    batch = pos_h.shape[0]
    p1 = _make_phase1(batch, ent_emb.shape[0])
    p2 = _make_phase2(batch, rel_emb.shape[0])
    tail_base = (ent_emb.shape[0] // _SUB) * _SUB
    gathered = p1(pos_h.astype(jnp.int32), pos_t.astype(jnp.int32),
                  neg_h.astype(jnp.int32), neg_t.astype(jnp.int32),
                  ent_emb.T, ent_emb[tail_base:])
    partials = p2(gathered, pos_r.astype(jnp.int32), rel_emb.reshape(-1))
    return jnp.sum(partials) / batch


# speedup vs baseline: 2.0585x; 2.0585x over previous
import functools

import jax
import jax.numpy as jnp
from jax import lax
from jax.experimental import pallas as pl
from jax.experimental.pallas import tpu as pltpu
from jax.experimental.pallas import tpu_sc as plsc

_MARGIN = 3.0
_DIM = 64
_LANES = 16
_CHUNK = 128


def _make_sc_kernel(batch):
    info = plsc.get_sparse_core_info()
    nw = info.num_cores * info.num_subcores
    per_w = batch // nw
    n_chunks = per_w // _CHUNK
    mesh = plsc.VectorSubcoreMesh(core_axis_name="c", subcore_axis_name="s")

    @functools.partial(
        pl.kernel,
        mesh=mesh,
        out_type=jax.ShapeDtypeStruct((nw, _LANES), jnp.float32),
        compiler_params=pltpu.CompilerParams(needs_layout_passes=False),
        scratch_types=[
            pltpu.VMEM((per_w,), jnp.int32),
            pltpu.VMEM((per_w,), jnp.int32),
            pltpu.VMEM((per_w,), jnp.int32),
            pltpu.VMEM((per_w,), jnp.int32),
            pltpu.VMEM((per_w,), jnp.int32),
            pltpu.VMEM((_CHUNK, _DIM), jnp.float32),
            pltpu.VMEM((_CHUNK, _DIM), jnp.float32),
            pltpu.VMEM((_CHUNK, _DIM), jnp.float32),
            pltpu.VMEM((_CHUNK, _DIM), jnp.float32),
            pltpu.VMEM((_CHUNK, _DIM), jnp.float32),
            pltpu.VMEM((_LANES,), jnp.float32),
            pltpu.SemaphoreType.DMA,
        ],
    )
    def trans_e(ph_hbm, pr_hbm, pt_hbm, nh_hbm, nt_hbm, ent_hbm, rel_hbm,
                out_hbm, ph_i, pr_i, pt_i, nh_i, nt_i,
                h_rows, r_rows, t_rows, nh_rows, nt_rows, part_v, sem):
        wid = lax.axis_index("s") * info.num_cores + lax.axis_index("c")
        zero16 = jnp.zeros((_LANES,), jnp.float32)

        wbase = wid * per_w
        pltpu.sync_copy(ph_hbm.at[pl.ds(wbase, per_w)], ph_i)
        pltpu.sync_copy(pr_hbm.at[pl.ds(wbase, per_w)], pr_i)
        pltpu.sync_copy(pt_hbm.at[pl.ds(wbase, per_w)], pt_i)
        pltpu.sync_copy(nh_hbm.at[pl.ds(wbase, per_w)], nh_i)
        pltpu.sync_copy(nt_hbm.at[pl.ds(wbase, per_w)], nt_i)

        def chunk_body(c, part):
            def fire_body(g, carry):
                base16 = pl.ds(c * _CHUNK + g * _LANES, _LANES)
                phv, prv = ph_i[base16], pr_i[base16]
                ptv, nhv, ntv = pt_i[base16], nh_i[base16], nt_i[base16]
                for u in range(_LANES):
                    j = g * _LANES + u
                    pltpu.async_copy(ent_hbm.at[phv[u]], h_rows.at[j], sem)
                    pltpu.async_copy(rel_hbm.at[prv[u]], r_rows.at[j], sem)
                    pltpu.async_copy(ent_hbm.at[ptv[u]], t_rows.at[j], sem)
                    pltpu.async_copy(ent_hbm.at[nhv[u]], nh_rows.at[j], sem)
                    pltpu.async_copy(ent_hbm.at[ntv[u]], nt_rows.at[j], sem)
                return carry

            lax.fori_loop(0, _CHUNK // _LANES, fire_body, 0)
            for buf in (h_rows, r_rows, t_rows, nh_rows, nt_rows):
                pltpu.make_async_copy(
                    ent_hbm.at[pl.ds(0, _CHUNK)], buf, sem).wait()

            def elem_body(j, acc):
                d = zero16
                for u in range(_DIM // _LANES):
                    s = pl.ds(u * _LANES, _LANES)
                    d = d + (jnp.abs(h_rows[j, s] + r_rows[j, s]
                                     - t_rows[j, s])
                             - jnp.abs(nh_rows[j, s] + r_rows[j, s]
                                       - nt_rows[j, s]))
                return acc + jnp.maximum(jnp.sum(d) + _MARGIN, 0.0)

            return lax.fori_loop(0, _CHUNK, elem_body, part)

        part = lax.fori_loop(0, n_chunks, chunk_body, jnp.float32(0.0))
        part_v[...] = zero16 + part * (1.0 / 16.0)
        pltpu.sync_copy(part_v, out_hbm.at[wid])

    return trans_e


@jax.jit
def kernel(pos_h, pos_r, pos_t, neg_h, neg_t, ent_emb, rel_emb):
    batch = pos_h.shape[0]
    sc_fn = _make_sc_kernel(batch)
    partials = sc_fn(pos_h.astype(jnp.int32), pos_r.astype(jnp.int32),
                     pos_t.astype(jnp.int32), neg_h.astype(jnp.int32),
                     neg_t.astype(jnp.int32), ent_emb, rel_emb)
    return jnp.sum(partials) / batch


# speedup vs baseline: 2.0806x; 1.0108x over previous
import functools

import jax
import jax.numpy as jnp
from jax import lax
from jax.experimental import pallas as pl
from jax.experimental.pallas import tpu as pltpu
from jax.experimental.pallas import tpu_sc as plsc

_MARGIN = 3.0
_DIM = 64
_LANES = 16
_CHUNK = 128


def _make_sc_kernel(batch):
    info = plsc.get_sparse_core_info()
    nw = info.num_cores * info.num_subcores
    per_w = batch // nw
    n_chunks = per_w // _CHUNK
    mesh = plsc.VectorSubcoreMesh(core_axis_name="c", subcore_axis_name="s")

    @functools.partial(
        pl.kernel,
        mesh=mesh,
        out_type=jax.ShapeDtypeStruct((nw, _LANES), jnp.float32),
        compiler_params=pltpu.CompilerParams(needs_layout_passes=False),
        scratch_types=[
            pltpu.VMEM((per_w,), jnp.int32),
            pltpu.VMEM((per_w,), jnp.int32),
            pltpu.VMEM((per_w,), jnp.int32),
            pltpu.VMEM((per_w,), jnp.int32),
            pltpu.VMEM((per_w,), jnp.int32),
            pltpu.VMEM((2, _CHUNK * _DIM), jnp.float32),
            pltpu.VMEM((2, _CHUNK * _DIM), jnp.float32),
            pltpu.VMEM((2, _CHUNK * _DIM), jnp.float32),
            pltpu.VMEM((2, _CHUNK * _DIM), jnp.float32),
            pltpu.VMEM((2, _CHUNK * _DIM), jnp.float32),
            pltpu.VMEM((_LANES,), jnp.float32),
            pltpu.SemaphoreType.DMA,
            pltpu.SemaphoreType.DMA,
        ],
    )
    def trans_e(ph_hbm, pr_hbm, pt_hbm, nh_hbm, nt_hbm, ent_hbm, rel_hbm,
                dummy_hbm, out_hbm, ph_i, pr_i, pt_i, nh_i, nt_i,
                h_rows, r_rows, t_rows, nh_rows, nt_rows, part_v,
                sem0, sem1):
        wid = lax.axis_index("s") * info.num_cores + lax.axis_index("c")
        zero16 = jnp.zeros((_LANES,), jnp.float32)

        wbase = wid * per_w
        pltpu.sync_copy(ph_hbm.at[pl.ds(wbase, per_w)], ph_i)
        pltpu.sync_copy(pr_hbm.at[pl.ds(wbase, per_w)], pr_i)
        pltpu.sync_copy(pt_hbm.at[pl.ds(wbase, per_w)], pt_i)
        pltpu.sync_copy(nh_hbm.at[pl.ds(wbase, per_w)], nh_i)
        pltpu.sync_copy(nt_hbm.at[pl.ds(wbase, per_w)], nt_i)

        def fire_chunk(c, slot):
            sem = sem0 if slot == 0 else sem1

            def fire_body(g, carry):
                base16 = pl.ds(c * _CHUNK + g * _LANES, _LANES)
                phv, prv = ph_i[base16], pr_i[base16]
                ptv, nhv, ntv = pt_i[base16], nh_i[base16], nt_i[base16]
                for u in range(_LANES):
                    dst = pl.ds((g * _LANES + u) * _DIM, _DIM)
                    pltpu.async_copy(ent_hbm.at[phv[u]],
                                     h_rows.at[slot, dst], sem)
                    pltpu.async_copy(rel_hbm.at[prv[u]],
                                     r_rows.at[slot, dst], sem)
                    pltpu.async_copy(ent_hbm.at[ptv[u]],
                                     t_rows.at[slot, dst], sem)
                    pltpu.async_copy(ent_hbm.at[nhv[u]],
                                     nh_rows.at[slot, dst], sem)
                    pltpu.async_copy(ent_hbm.at[ntv[u]],
                                     nt_rows.at[slot, dst], sem)
                return carry

            lax.fori_loop(0, _CHUNK // _LANES, fire_body, 0)

        def drain_chunk(slot):
            sem = sem0 if slot == 0 else sem1
            for buf in (h_rows, r_rows, t_rows, nh_rows, nt_rows):
                pltpu.make_async_copy(dummy_hbm, buf.at[slot], sem).wait()

        def compute_chunk(slot, part):
            def elem_body(j, acc):
                d = zero16
                base = j * _DIM
                for u in range(_DIM // _LANES):
                    s = pl.ds(base + u * _LANES, _LANES)
                    d = d + (jnp.abs(h_rows[slot, s] + r_rows[slot, s]
                                     - t_rows[slot, s])
                             - jnp.abs(nh_rows[slot, s] + r_rows[slot, s]
                                       - nt_rows[slot, s]))
                return acc + jnp.maximum(jnp.sum(d) + _MARGIN, 0.0)

            return lax.fori_loop(0, _CHUNK, elem_body, part)

        fire_chunk(0, 0)
        part = jnp.float32(0.0)
        for c in range(n_chunks):
            slot = c % 2
            if c + 1 < n_chunks:
                fire_chunk(c + 1, (c + 1) % 2)
            drain_chunk(slot)
            part = compute_chunk(slot, part)
        part_v[...] = zero16 + part * (1.0 / 16.0)
        pltpu.sync_copy(part_v, out_hbm.at[wid])

    return trans_e


@jax.jit
def kernel(pos_h, pos_r, pos_t, neg_h, neg_t, ent_emb, rel_emb):
    batch = pos_h.shape[0]
    sc_fn = _make_sc_kernel(batch)
    dummy = jax.lax.slice(rel_emb.reshape(-1), (0,), (_CHUNK * _DIM,))
    partials = sc_fn(pos_h.astype(jnp.int32), pos_r.astype(jnp.int32),
                     pos_t.astype(jnp.int32), neg_h.astype(jnp.int32),
                     neg_t.astype(jnp.int32), ent_emb, rel_emb, dummy)
    return jnp.sum(partials) / batch
